# baseline (speedup 1.0000x reference)
import functools

import jax
import jax.numpy as jnp
from jax import lax
from jax.experimental import pallas as pl
from jax.experimental.pallas import tpu as pltpu
from jax.experimental.pallas import tpu_sc as plsc

N = 10000
NP = 10240
E = 320000
D_IN = 128
CHUNK = 128
NW = 32
ROWS_PER_W = 80
NB = 5
LOOKAHEAD = 3
SLACK = 1
SLAB = 88
EP_ROWS = NW * ROWS_PER_W + 8
EP = EP_ROWS * CHUNK
SUB_ROWS = NP // 16
CW = 16

_mesh = plsc.VectorSubcoreMesh(core_axis_name="c", subcore_axis_name="s")
_sc_params = pltpu.CompilerParams(use_tc_tiling_on_sc=False)


def _seg_body(ytab_h, src_h, dst_h, z_h, part_h, src_v, dst_v, rows_v, acc_s,
              sem_g, sem_s, zc_h=None, on_h=None, cpart_h=None, ones_v=None,
              cacc_s=None, sem_c=None):
    cid = lax.axis_index("c")
    sid = lax.axis_index("s")
    wid = cid * 16 + sid
    base = wid * ROWS_PER_W
    with_cnt = cpart_h is not None

    def rbuf(b):
        return rows_v.at[pl.ds(b * CHUNK, CHUNK)]

    pltpu.sync_copy(src_h.at[pl.ds(base, SLAB)], src_v)
    pltpu.sync_copy(dst_h.at[pl.ds(base, ROWS_PER_W)], dst_v)
    pltpu.sync_copy(z_h, acc_s.at[pl.ds(sid * SUB_ROWS, SUB_ROWS)])
    if with_cnt:
        pltpu.sync_copy(on_h, ones_v)
        pltpu.sync_copy(zc_h, cacc_s.at[pl.ds(sid * SUB_ROWS, SUB_ROWS)])
    plsc.subcore_barrier()

    for b in range(LOOKAHEAD):
        pltpu.async_copy(ytab_h.at[src_v.at[b]], rbuf(b), sem_g[b])

    @pl.loop(0, ROWS_PER_W // NB)
    def _(t):
        k0 = t * NB
        for b in range(NB):
            k = k0 + b
            pltpu.make_async_copy(ytab_h.at[src_v.at[k]], rbuf(b), sem_g[b]).wait()
            pltpu.async_copy(rbuf(b), acc_s.at[dst_v.at[k]], sem_s[b], add=True)
            if with_cnt:
                pltpu.async_copy(ones_v, cacc_s.at[dst_v.at[k]], sem_c[b], add=True)
            bs = (b - SLACK) % NB

            @pl.when(k >= SLACK)
            def _():
                pltpu.make_async_copy(
                    rbuf(bs), acc_s.at[dst_v.at[k - SLACK]], sem_s[bs]).wait()
                if with_cnt:
                    pltpu.make_async_copy(
                        ones_v, cacc_s.at[dst_v.at[k - SLACK]], sem_c[bs]).wait()

            bg = (b + LOOKAHEAD) % NB
            pltpu.async_copy(ytab_h.at[src_v.at[k + LOOKAHEAD]], rbuf(bg), sem_g[bg])

    for k in range(ROWS_PER_W - SLACK, ROWS_PER_W):
        b = k % NB
        pltpu.make_async_copy(rbuf(b), acc_s.at[dst_v.at[k]], sem_s[b]).wait()
        if with_cnt:
            pltpu.make_async_copy(ones_v, cacc_s.at[dst_v.at[k]], sem_c[b]).wait()
    for k in range(ROWS_PER_W, ROWS_PER_W + LOOKAHEAD):
        b = k % NB
        pltpu.make_async_copy(ytab_h.at[src_v.at[k]], rbuf(b), sem_g[b]).wait()

    plsc.subcore_barrier()
    sl = pl.ds(sid * SUB_ROWS, SUB_ROWS)
    D = acc_s.shape[1]
    pltpu.sync_copy(acc_s.at[sl], part_h.at[sl, pl.ds(cid * D, D)])
    if with_cnt:
        pltpu.sync_copy(cacc_s.at[sl], cpart_h.at[sl, pl.ds(cid * CW, CW)])


def _seg_sum_sc(ytab, src2, dst2, with_count):
    D = ytab.shape[1]
    f32 = jnp.float32
    zeros_d = jnp.zeros((SUB_ROWS, D), f32)
    scratch = [
        pltpu.VMEM((SLAB, CHUNK), jnp.int32),
        pltpu.VMEM((ROWS_PER_W, CHUNK), jnp.int32),
        pltpu.VMEM((NB * CHUNK, D), f32),
        pltpu.VMEM_SHARED((NP, D), f32),
        [pltpu.SemaphoreType.DMA] * NB,
        [pltpu.SemaphoreType.DMA] * NB,
    ]
    if with_count:
        scratch += [pltpu.VMEM((CHUNK, CW), f32), pltpu.VMEM_SHARED((NP, CW), f32),
                    [pltpu.SemaphoreType.DMA] * NB]

        @functools.partial(
            pl.kernel, mesh=_mesh, compiler_params=_sc_params,
            out_type=(jax.ShapeDtypeStruct((NP, 2 * D), f32),
                      jax.ShapeDtypeStruct((NP, 2 * CW), f32)),
            scratch_types=scratch)
        def k(ytab_h, src_h, dst_h, z_h, zc_h, on_h, part_h, cpart_h,
              src_v, dst_v, rows_v, acc_s, sem_g, sem_s, ones_v, cacc_s, sem_c):
            _seg_body(ytab_h, src_h, dst_h, z_h, part_h, src_v, dst_v, rows_v,
                      acc_s, sem_g, sem_s, zc_h, on_h, cpart_h, ones_v, cacc_s,
                      sem_c)

        return k(ytab, src2, dst2, zeros_d,
                 jnp.zeros((SUB_ROWS, CW), f32), jnp.ones((CHUNK, CW), f32))

    @functools.partial(
        pl.kernel, mesh=_mesh, compiler_params=_sc_params,
        out_type=jax.ShapeDtypeStruct((NP, 2 * D), f32),
        scratch_types=scratch)
    def k(ytab_h, src_h, dst_h, z_h, part_h, src_v, dst_v, rows_v, acc_s,
          sem_g, sem_s):
        _seg_body(ytab_h, src_h, dst_h, z_h, part_h, src_v, dst_v, rows_v,
                  acc_s, sem_g, sem_s)

    return k(ytab, src2, dst2, zeros_d)


_DNUMS = (((1,), (1,)), ((), ()))


def _matmul(x, w):
    BR = 1000

    def kern(x_ref, w_ref, o_ref):
        o_ref[...] = lax.dot_general(
            x_ref[...], w_ref[...], _DNUMS,
            preferred_element_type=jnp.float32, precision=lax.Precision.HIGHEST)

    return pl.pallas_call(
        kern,
        grid=(N // BR,),
        in_specs=[pl.BlockSpec((BR, x.shape[1]), lambda i: (i, 0)),
                  pl.BlockSpec(w.shape, lambda i: (0, 0))],
        out_specs=pl.BlockSpec((BR, w.shape[0]), lambda i: (i, 0)),
        out_shape=jax.ShapeDtypeStruct((NP, w.shape[0]), jnp.float32),
    )(x, w)


def _combine(p, c, xr, b, g=None, be=None, Wnl=None, Wnr=None):
    BR = 1280
    D = p.shape[1] // 2
    with_ln = Wnl is not None
    b2 = b.reshape(1, D)
    ins = [p, c, xr, b2]
    in_specs = [
        pl.BlockSpec((BR, 2 * D), lambda i: (i, 0)),
        pl.BlockSpec((BR, 2 * CW), lambda i: (i, 0)),
        pl.BlockSpec((BR, D), lambda i: (i, 0)),
        pl.BlockSpec((1, D), lambda i: (0, 0)),
    ]
    if with_ln:
        Dn = Wnl.shape[0]
        ins += [g.reshape(1, D), be.reshape(1, D), Wnl, Wnr]
        in_specs += [pl.BlockSpec((1, D), lambda i: (0, 0)),
                     pl.BlockSpec((1, D), lambda i: (0, 0)),
                     pl.BlockSpec(Wnl.shape, lambda i: (0, 0)),
                     pl.BlockSpec(Wnr.shape, lambda i: (0, 0))]
        out_shape = (jax.ShapeDtypeStruct((NP, Dn), jnp.float32),
                     jax.ShapeDtypeStruct((NP, Dn), jnp.float32))
        out_specs = (pl.BlockSpec((BR, Dn), lambda i: (i, 0)),
                     pl.BlockSpec((BR, Dn), lambda i: (i, 0)))
    else:
        out_shape = jax.ShapeDtypeStruct((N, D), jnp.float32)
        out_specs = pl.BlockSpec((BR, D), lambda i: (i, 0))

    def kern(p_ref, c_ref, xr_ref, b_ref, *rest):
        if with_ln:
            g_ref, be_ref, wnl_ref, wnr_ref, y_ref, yr_ref = rest
        else:
            (o_ref,) = rest
        pb = p_ref[...]
        cb = c_ref[...]
        agg = pb[:, :D] + pb[:, D:]
        cnt = cb[:, :1] + cb[:, CW:CW + 1]
        inv = 1.0 / jnp.maximum(cnt, 1.0)
        out = agg * inv + b_ref[...] + xr_ref[...]
        nrm = jnp.sqrt(jnp.sum(out * out, axis=-1, keepdims=True))
        out = out / jnp.maximum(nrm, 1e-12)
        if with_ln:
            mu = jnp.mean(out, axis=-1, keepdims=True)
            var = jnp.mean((out - mu) ** 2, axis=-1, keepdims=True)
            hn = (out - mu) / jnp.sqrt(var + 1e-5) * g_ref[...] + be_ref[...]
            hn = jnp.maximum(hn, 0.0)
            y_ref[...] = lax.dot_general(
                hn, wnl_ref[...], _DNUMS, preferred_element_type=jnp.float32, precision=lax.Precision.HIGHEST)
            yr_ref[...] = lax.dot_general(
                hn, wnr_ref[...], _DNUMS, preferred_element_type=jnp.float32, precision=lax.Precision.HIGHEST)
        else:
            o_ref[...] = out

    return pl.pallas_call(
        kern,
        grid=(NP // BR,),
        in_specs=in_specs,
        out_specs=out_specs,
        out_shape=out_shape,
    )(*ins)


def kernel(x, edge_index, W1l, b1l, W1r, g1, be1, W2l, b2l, W2r, g2, be2, W3l, b3l, W3r):
    src = edge_index[0]
    dst = edge_index[1]
    pad_e = EP - E
    pad_iota = jnp.arange(pad_e, dtype=jnp.int32)
    src2 = jnp.concatenate([src, pad_iota % N]).reshape(EP_ROWS, CHUNK)
    dst2 = jnp.concatenate([dst, N + pad_iota % (NP - N)]).reshape(EP_ROWS, CHUNK)

    y1 = _matmul(x, W1l)
    xr1 = _matmul(x, W1r)
    p1, cnt = _seg_sum_sc(y1, src2, dst2, True)
    y2, xr2 = _combine(p1, cnt, xr1, b1l, g1, be1, W2l, W2r)
    p2 = _seg_sum_sc(y2, src2, dst2, False)
    y3, xr3 = _combine(p2, cnt, xr2, b2l, g2, be2, W3l, W3r)
    p3 = _seg_sum_sc(y3, src2, dst2, False)
    return _combine(p3, cnt, xr3, b3l)


# speedup vs baseline: 1.2035x; 1.2035x over previous
import functools

import jax
import jax.numpy as jnp
from jax import lax
from jax.experimental import pallas as pl
from jax.experimental.pallas import tpu as pltpu
from jax.experimental.pallas import tpu_sc as plsc

N = 10000
NP = 10240
E = 320000
D_IN = 128
CHUNK = 128
NW = 32
ROWS_PER_W = 80
NB_CNT, LA_CNT, SLK_CNT = 5, 3, 1
NB_PLAIN, LA_PLAIN, SLK_PLAIN = 8, 4, 3
SLAB = 88
EP_ROWS = NW * ROWS_PER_W + 8
EP = EP_ROWS * CHUNK
SUB_ROWS = NP // 16
CW = 16

_mesh = plsc.VectorSubcoreMesh(core_axis_name="c", subcore_axis_name="s")
_sc_params = pltpu.CompilerParams(use_tc_tiling_on_sc=False)


def _seg_body(ytab_h, src_h, dst_h, z_h, part_h, src_v, dst_v, rows_v, acc_s,
              sem_g, sem_s, zc_h=None, on_h=None, cpart_h=None, ones_v=None,
              cacc_s=None, sem_c=None):
    cid = lax.axis_index("c")
    sid = lax.axis_index("s")
    wid = cid * 16 + sid
    base = wid * ROWS_PER_W
    with_cnt = cpart_h is not None
    nb = len(sem_g)
    la, slk = (LA_CNT, SLK_CNT) if with_cnt else (LA_PLAIN, SLK_PLAIN)

    def rbuf(b):
        return rows_v.at[pl.ds(b * CHUNK, CHUNK)]

    zsl = pl.ds(sid * SUB_ROWS, SUB_ROWS)
    pro = [pltpu.async_copy(src_h.at[pl.ds(base, SLAB)], src_v, sem_g[0]),
           pltpu.async_copy(dst_h.at[pl.ds(base, ROWS_PER_W)], dst_v, sem_g[1]),
           pltpu.async_copy(z_h, acc_s.at[zsl], sem_g[2])]
    if with_cnt:
        pro += [pltpu.async_copy(on_h, ones_v, sem_s[0]),
                pltpu.async_copy(zc_h, cacc_s.at[zsl], sem_s[1])]
    for c in pro:
        c.wait()
    plsc.subcore_barrier()

    for b in range(la):
        pltpu.async_copy(ytab_h.at[src_v.at[b]], rbuf(b), sem_g[b])

    @pl.loop(0, ROWS_PER_W // nb)
    def _(t):
        k0 = t * nb
        for b in range(nb):
            k = k0 + b
            pltpu.make_async_copy(ytab_h.at[src_v.at[k]], rbuf(b), sem_g[b]).wait()
            pltpu.async_copy(rbuf(b), acc_s.at[dst_v.at[k]], sem_s[b], add=True)
            if with_cnt:
                pltpu.async_copy(ones_v, cacc_s.at[dst_v.at[k]], sem_c[b], add=True)
            bs = (b - slk) % nb

            @pl.when(k >= slk)
            def _():
                pltpu.make_async_copy(
                    rbuf(bs), acc_s.at[dst_v.at[k - slk]], sem_s[bs]).wait()
                if with_cnt:
                    pltpu.make_async_copy(
                        ones_v, cacc_s.at[dst_v.at[k - slk]], sem_c[bs]).wait()

            bg = (b + la) % nb
            pltpu.async_copy(ytab_h.at[src_v.at[k + la]], rbuf(bg), sem_g[bg])

    for k in range(ROWS_PER_W - slk, ROWS_PER_W):
        b = k % nb
        pltpu.make_async_copy(rbuf(b), acc_s.at[dst_v.at[k]], sem_s[b]).wait()
        if with_cnt:
            pltpu.make_async_copy(ones_v, cacc_s.at[dst_v.at[k]], sem_c[b]).wait()
    for k in range(ROWS_PER_W, ROWS_PER_W + la):
        b = k % nb
        pltpu.make_async_copy(ytab_h.at[src_v.at[k]], rbuf(b), sem_g[b]).wait()

    plsc.subcore_barrier()
    sl = pl.ds(sid * SUB_ROWS, SUB_ROWS)
    pltpu.sync_copy(acc_s.at[sl], part_h.at[pl.ds(cid * NP + sid * SUB_ROWS, SUB_ROWS)])
    if with_cnt:
        pltpu.sync_copy(cacc_s.at[sl], cpart_h.at[pl.ds(cid * NP + sid * SUB_ROWS, SUB_ROWS)])


def _seg_sum_sc(ytab, src2, dst2, with_count):
    D = ytab.shape[1]
    f32 = jnp.float32
    nb = NB_CNT if with_count else NB_PLAIN
    zeros_d = jnp.zeros((SUB_ROWS, D), f32)
    scratch = [
        pltpu.VMEM((SLAB, CHUNK), jnp.int32),
        pltpu.VMEM((ROWS_PER_W, CHUNK), jnp.int32),
        pltpu.VMEM((nb * CHUNK, D), f32),
        pltpu.VMEM_SHARED((NP, D), f32),
        [pltpu.SemaphoreType.DMA] * nb,
        [pltpu.SemaphoreType.DMA] * nb,
    ]
    if with_count:
        scratch += [pltpu.VMEM((CHUNK, CW), f32), pltpu.VMEM_SHARED((NP, CW), f32),
                    [pltpu.SemaphoreType.DMA] * nb]

        @functools.partial(
            pl.kernel, mesh=_mesh, compiler_params=_sc_params,
            out_type=(jax.ShapeDtypeStruct((2 * NP, D), f32),
                      jax.ShapeDtypeStruct((2 * NP, CW), f32)),
            scratch_types=scratch)
        def k(ytab_h, src_h, dst_h, z_h, zc_h, on_h, part_h, cpart_h,
              src_v, dst_v, rows_v, acc_s, sem_g, sem_s, ones_v, cacc_s, sem_c):
            _seg_body(ytab_h, src_h, dst_h, z_h, part_h, src_v, dst_v, rows_v,
                      acc_s, sem_g, sem_s, zc_h, on_h, cpart_h, ones_v, cacc_s,
                      sem_c)

        return k(ytab, src2, dst2, zeros_d,
                 jnp.zeros((SUB_ROWS, CW), f32), jnp.ones((CHUNK, CW), f32))

    @functools.partial(
        pl.kernel, mesh=_mesh, compiler_params=_sc_params,
        out_type=jax.ShapeDtypeStruct((2 * NP, D), f32),
        scratch_types=scratch)
    def k(ytab_h, src_h, dst_h, z_h, part_h, src_v, dst_v, rows_v, acc_s,
          sem_g, sem_s):
        _seg_body(ytab_h, src_h, dst_h, z_h, part_h, src_v, dst_v, rows_v,
                  acc_s, sem_g, sem_s)

    return k(ytab, src2, dst2, zeros_d)


_DNUMS = (((1,), (1,)), ((), ()))


def _matmul(x, w):
    BR = 1000

    def kern(x_ref, w_ref, o_ref):
        o_ref[...] = lax.dot_general(
            x_ref[...], w_ref[...], _DNUMS,
            preferred_element_type=jnp.float32, precision=lax.Precision.HIGHEST)

    return pl.pallas_call(
        kern,
        grid=(N // BR,),
        in_specs=[pl.BlockSpec((BR, x.shape[1]), lambda i: (i, 0)),
                  pl.BlockSpec(w.shape, lambda i: (0, 0))],
        out_specs=pl.BlockSpec((BR, w.shape[0]), lambda i: (i, 0)),
        out_shape=jax.ShapeDtypeStruct((NP, w.shape[0]), jnp.float32),
    )(x, w)


def _combine(p, c, xr, b, g=None, be=None, Wnl=None, Wnr=None):
    BR = 1280
    NBLK = NP // BR
    D = p.shape[1]
    with_ln = Wnl is not None
    b2 = b.reshape(1, D)
    ins = [p, p, c, c, xr, b2]
    in_specs = [
        pl.BlockSpec((BR, D), lambda i: (i, 0)),
        pl.BlockSpec((BR, D), lambda i: (i + NBLK, 0)),
        pl.BlockSpec((BR, CW), lambda i: (i, 0)),
        pl.BlockSpec((BR, CW), lambda i: (i + NBLK, 0)),
        pl.BlockSpec((BR, D), lambda i: (i, 0)),
        pl.BlockSpec((1, D), lambda i: (0, 0)),
    ]
    if with_ln:
        Dn = Wnl.shape[0]
        ins += [g.reshape(1, D), be.reshape(1, D), Wnl, Wnr]
        in_specs += [pl.BlockSpec((1, D), lambda i: (0, 0)),
                     pl.BlockSpec((1, D), lambda i: (0, 0)),
                     pl.BlockSpec(Wnl.shape, lambda i: (0, 0)),
                     pl.BlockSpec(Wnr.shape, lambda i: (0, 0))]
        out_shape = (jax.ShapeDtypeStruct((NP, Dn), jnp.float32),
                     jax.ShapeDtypeStruct((NP, Dn), jnp.float32))
        out_specs = (pl.BlockSpec((BR, Dn), lambda i: (i, 0)),
                     pl.BlockSpec((BR, Dn), lambda i: (i, 0)))
    else:
        out_shape = jax.ShapeDtypeStruct((N, D), jnp.float32)
        out_specs = pl.BlockSpec((BR, D), lambda i: (i, 0))

    def kern(p0_ref, p1_ref, c0_ref, c1_ref, xr_ref, b_ref, *rest):
        if with_ln:
            g_ref, be_ref, wnl_ref, wnr_ref, y_ref, yr_ref = rest
        else:
            (o_ref,) = rest
        agg = p0_ref[...] + p1_ref[...]
        cnt = c0_ref[...][:, :1] + c1_ref[...][:, :1]
        inv = 1.0 / jnp.maximum(cnt, 1.0)
        out = agg * inv + b_ref[...] + xr_ref[...]
        nrm = jnp.sqrt(jnp.sum(out * out, axis=-1, keepdims=True))
        out = out / jnp.maximum(nrm, 1e-12)
        if with_ln:
            mu = jnp.mean(out, axis=-1, keepdims=True)
            var = jnp.mean((out - mu) ** 2, axis=-1, keepdims=True)
            hn = (out - mu) / jnp.sqrt(var + 1e-5) * g_ref[...] + be_ref[...]
            hn = jnp.maximum(hn, 0.0)
            y_ref[...] = lax.dot_general(
                hn, wnl_ref[...], _DNUMS, preferred_element_type=jnp.float32, precision=lax.Precision.HIGHEST)
            yr_ref[...] = lax.dot_general(
                hn, wnr_ref[...], _DNUMS, preferred_element_type=jnp.float32, precision=lax.Precision.HIGHEST)
        else:
            o_ref[...] = out

    return pl.pallas_call(
        kern,
        grid=(NP // BR,),
        in_specs=in_specs,
        out_specs=out_specs,
        out_shape=out_shape,
    )(*ins)


def kernel(x, edge_index, W1l, b1l, W1r, g1, be1, W2l, b2l, W2r, g2, be2, W3l, b3l, W3r):
    src = edge_index[0]
    dst = edge_index[1]
    pad_e = EP - E
    pad_iota = jnp.arange(pad_e, dtype=jnp.int32)
    src2 = jnp.concatenate([src, pad_iota % N]).reshape(EP_ROWS, CHUNK)
    dst2 = jnp.concatenate([dst, N + pad_iota % (NP - N)]).reshape(EP_ROWS, CHUNK)

    y1 = _matmul(x, W1l)
    xr1 = _matmul(x, W1r)
    p1, cnt = _seg_sum_sc(y1, src2, dst2, True)
    y2, xr2 = _combine(p1, cnt, xr1, b1l, g1, be1, W2l, W2r)
    p2 = _seg_sum_sc(y2, src2, dst2, False)
    y3, xr3 = _combine(p2, cnt, xr2, b2l, g2, be2, W3l, W3r)
    p3 = _seg_sum_sc(y3, src2, dst2, False)
    return _combine(p3, cnt, xr3, b3l)
